# baseline (speedup 1.0000x reference)
import functools

import jax
import jax.numpy as jnp
from jax import lax
from jax.experimental import pallas as pl
from jax.experimental.pallas import tpu as pltpu
from jax.experimental.pallas import tpu_sc as plsc

COLUMN_COUNT = 65536
CELLS_PER_COLUMN = 32
NUM_CELLS = COLUMN_COUNT * CELLS_PER_COLUMN

NUM_WORKERS = 32
COLS_PER_W = COLUMN_COUNT // NUM_WORKERS
CELLS_PER_W = COLS_PER_W * CELLS_PER_COLUMN
LANES = 16

NCHUNK = 2
CHUNK_COLS = COLS_PER_W // NCHUNK
CHUNK_CELLS = CHUNK_COLS * CELLS_PER_COLUMN

_mesh = plsc.VectorSubcoreMesh(core_axis_name="c", subcore_axis_name="s")


@functools.partial(
    pl.kernel,
    mesh=_mesh,
    compiler_params=pltpu.CompilerParams(
        needs_layout_passes=False,
        disable_bounds_checks=True,
        skip_device_barrier=True,
    ),
    out_type=jax.ShapeDtypeStruct((NUM_CELLS,), jnp.float32),
    scratch_types=[
        pltpu.VMEM((COLS_PER_W,), jnp.float32),
        pltpu.VMEM((CHUNK_CELLS,), jnp.float32),
        pltpu.VMEM((CHUNK_CELLS,), jnp.float32),
        pltpu.SemaphoreType.DMA,
        pltpu.SemaphoreType.DMA,
        pltpu.SemaphoreType.DMA,
        pltpu.SemaphoreType.DMA,
    ],
)
def _sc_burst(cols_hbm, act_hbm, in_v, buf0, buf1, sem_in, sem_in2, sem0, sem1):
    wid = lax.axis_index("s") * 2 + lax.axis_index("c")
    col_base = wid * COLS_PER_W
    cell_base = wid * CELLS_PER_W

    in_dma0 = pltpu.async_copy(
        cols_hbm.at[pl.ds(col_base, CHUNK_COLS)],
        in_v.at[pl.ds(0, CHUNK_COLS)],
        sem_in,
    )
    in_dma1 = pltpu.async_copy(
        cols_hbm.at[pl.ds(col_base + CHUNK_COLS, COLS_PER_W - CHUNK_COLS)],
        in_v.at[pl.ds(CHUNK_COLS, COLS_PER_W - CHUNK_COLS)],
        sem_in2,
    )
    in_dma0.wait()

    ones16 = jnp.full((LANES,), 1.0, jnp.float32)
    zeros16 = jnp.zeros((LANES,), jnp.float32)
    lane_idx = [jnp.full((LANES, 1), l, jnp.int32) for l in range(LANES)]
    _gdn = lax.GatherDimensionNumbers(
        offset_dims=(), collapsed_slice_dims=(0,), start_index_map=(0,)
    )

    def _lane_bcast(vec, l):
        return lax.gather(
            vec,
            lane_idx[l],
            dimension_numbers=_gdn,
            slice_sizes=(1,),
            mode=lax.GatherScatterMode.PROMISE_IN_BOUNDS,
        )

    bufs = (buf0, buf1)
    sems = (sem0, sem1)
    dmas = [None, None]
    for k in range(NCHUNK):
        if k == 1:
            in_dma1.wait()
        b = k & 1
        if dmas[b] is not None:
            dmas[b].wait()
        buf = bufs[b]

        @plsc.parallel_loop(0, CHUNK_COLS // LANES, unroll=2)
        def _expand(i, _k=k, _buf=buf):
            v = in_v[pl.ds(_k * CHUNK_COLS + i * LANES, LANES)]
            m = jnp.where(v > 0.0, ones16, zeros16)
            base = i * (LANES * CELLS_PER_COLUMN)
            for l in range(LANES):
                mv = _lane_bcast(m, l)
                _buf[pl.ds(base + l * CELLS_PER_COLUMN, LANES)] = mv
                _buf[pl.ds(base + l * CELLS_PER_COLUMN + LANES, LANES)] = mv

        dmas[b] = pltpu.async_copy(
            buf,
            act_hbm.at[pl.ds(cell_base + k * CHUNK_CELLS, CHUNK_CELLS)],
            sems[b],
        )

    dmas[0].wait()
    dmas[1].wait()


_ZBLOCK = NUM_CELLS // 8


def _tc_zero_body(o_ref):
    o_ref[...] = jnp.zeros_like(o_ref)


_tc_zeros = pl.pallas_call(
    _tc_zero_body,
    out_shape=jax.ShapeDtypeStruct((NUM_CELLS,), jnp.float32),
    grid=(NUM_CELLS // _ZBLOCK,),
    out_specs=pl.BlockSpec((_ZBLOCK,), lambda i: (i,)),
    compiler_params=pltpu.CompilerParams(skip_device_barrier=True),
)


def kernel(active_columns):
    new_active = _sc_burst(active_columns)
    new_predictive = _tc_zeros()
    return (new_active, new_predictive)


# speedup vs baseline: 1.0118x; 1.0118x over previous
import functools

import jax
import jax.numpy as jnp
from jax import lax
from jax.experimental import pallas as pl
from jax.experimental.pallas import tpu as pltpu
from jax.experimental.pallas import tpu_sc as plsc

COLUMN_COUNT = 65536
CELLS_PER_COLUMN = 32
NUM_CELLS = COLUMN_COUNT * CELLS_PER_COLUMN

NUM_WORKERS = 32
COLS_PER_W = COLUMN_COUNT // NUM_WORKERS
CELLS_PER_W = COLS_PER_W * CELLS_PER_COLUMN
LANES = 16

NCHUNK = 4
CHUNK_COLS = COLS_PER_W // NCHUNK
CHUNK_CELLS = CHUNK_COLS * CELLS_PER_COLUMN

_mesh = plsc.VectorSubcoreMesh(core_axis_name="c", subcore_axis_name="s")


@functools.partial(
    pl.kernel,
    mesh=_mesh,
    compiler_params=pltpu.CompilerParams(
        needs_layout_passes=False,
        disable_bounds_checks=True,
        skip_device_barrier=True,
    ),
    out_type=jax.ShapeDtypeStruct((NUM_CELLS,), jnp.float32),
    scratch_types=[
        pltpu.VMEM((COLS_PER_W,), jnp.float32),
        pltpu.VMEM((CHUNK_CELLS,), jnp.float32),
        pltpu.VMEM((CHUNK_CELLS,), jnp.float32),
        pltpu.SemaphoreType.DMA,
        pltpu.SemaphoreType.DMA,
        pltpu.SemaphoreType.DMA,
        pltpu.SemaphoreType.DMA,
    ],
)
def _sc_burst(cols_hbm, act_hbm, in_v, buf0, buf1, sem_in, sem_in2, sem0, sem1):
    wid = lax.axis_index("s") * 2 + lax.axis_index("c")
    col_base = wid * COLS_PER_W
    cell_base = wid * CELLS_PER_W

    in_dma0 = pltpu.async_copy(
        cols_hbm.at[pl.ds(col_base, CHUNK_COLS)],
        in_v.at[pl.ds(0, CHUNK_COLS)],
        sem_in,
    )
    in_dma1 = pltpu.async_copy(
        cols_hbm.at[pl.ds(col_base + CHUNK_COLS, COLS_PER_W - CHUNK_COLS)],
        in_v.at[pl.ds(CHUNK_COLS, COLS_PER_W - CHUNK_COLS)],
        sem_in2,
    )
    in_dma0.wait()

    ones16 = jnp.full((LANES,), 1.0, jnp.float32)
    zeros16 = jnp.zeros((LANES,), jnp.float32)
    lane_idx = [jnp.full((LANES, 1), l, jnp.int32) for l in range(LANES)]
    _gdn = lax.GatherDimensionNumbers(
        offset_dims=(), collapsed_slice_dims=(0,), start_index_map=(0,)
    )

    def _lane_bcast(vec, l):
        return lax.gather(
            vec,
            lane_idx[l],
            dimension_numbers=_gdn,
            slice_sizes=(1,),
            mode=lax.GatherScatterMode.PROMISE_IN_BOUNDS,
        )

    bufs = (buf0, buf1)
    sems = (sem0, sem1)
    dmas = [None, None]
    for k in range(NCHUNK):
        if k == 1:
            in_dma1.wait()
        b = k & 1
        if dmas[b] is not None:
            dmas[b].wait()
        buf = bufs[b]

        @plsc.parallel_loop(0, CHUNK_COLS // LANES, unroll=2)
        def _expand(i, _k=k, _buf=buf):
            v = in_v[pl.ds(_k * CHUNK_COLS + i * LANES, LANES)]
            m = jnp.where(v > 0.0, ones16, zeros16)
            base = i * (LANES * CELLS_PER_COLUMN)
            for l in range(LANES):
                mv = _lane_bcast(m, l)
                _buf[pl.ds(base + l * CELLS_PER_COLUMN, LANES)] = mv
                _buf[pl.ds(base + l * CELLS_PER_COLUMN + LANES, LANES)] = mv

        dmas[b] = pltpu.async_copy(
            buf,
            act_hbm.at[pl.ds(cell_base + k * CHUNK_CELLS, CHUNK_CELLS)],
            sems[b],
        )

    dmas[0].wait()
    dmas[1].wait()


_ZBLOCK = NUM_CELLS // 8


def _tc_zero_body(o_ref):
    o_ref[...] = jnp.zeros_like(o_ref)


_tc_zeros = pl.pallas_call(
    _tc_zero_body,
    out_shape=jax.ShapeDtypeStruct((NUM_CELLS,), jnp.float32),
    grid=(NUM_CELLS // _ZBLOCK,),
    out_specs=pl.BlockSpec((_ZBLOCK,), lambda i: (i,)),
    compiler_params=pltpu.CompilerParams(skip_device_barrier=True),
)


def kernel(active_columns):
    new_active = _sc_burst(active_columns)
    new_predictive = _tc_zeros()
    return (new_active, new_predictive)
